# Initial kernel scaffold; baseline (speedup 1.0000x reference)
import jax
import jax.numpy as jnp
from jax.experimental import pallas as pl


def kernel(x, W_gate, w1, w3, w2):
    raise NotImplementedError("write your pallas kernel here")


# speedup vs baseline: 1.3152x; 1.3152x over previous
import functools

import jax
import jax.numpy as jnp
from jax.experimental import pallas as pl
from jax.experimental.pallas import tpu as pltpu

B, S, H, F, E, K = 2, 2048, 1024, 2048, 8, 2
T = B * S
TB = 4
TBS = T // TB


def _moe_dense_kernel(x_ref, wg_ref, w1_ref, w3_ref, w2_ref,
                      out_ref, bal_ref, z_ref,
                      combine_s, cnt_s, psum_s, lse2_s):
    tb = pl.program_id(0)
    e = pl.program_id(1)

    @pl.when(e == 0)
    def _router():
        xf = x_ref[...]
        logits = jax.lax.dot_general(
            xf, wg_ref[...],
            dimension_numbers=(((1,), (1,)), ((), ())),
            preferred_element_type=jnp.float32)
        m = jnp.max(logits, axis=-1, keepdims=True)
        ex = jnp.exp(logits - m)
        se = jnp.sum(ex, axis=-1, keepdims=True)
        probs = ex / se
        lane = jax.lax.broadcasted_iota(jnp.int32, (TBS, E), 1)
        p1 = jnp.max(probs, axis=-1, keepdims=True)
        a1 = jnp.min(jnp.where(probs == p1, lane, E), axis=-1, keepdims=True)
        oh1 = (lane == a1).astype(jnp.float32)
        pm = jnp.where(lane == a1, -1.0, probs)
        p2 = jnp.max(pm, axis=-1, keepdims=True)
        a2 = jnp.min(jnp.where(pm == p2, lane, E), axis=-1, keepdims=True)
        oh2 = (lane == a2).astype(jnp.float32)
        wsum = p1 + p2
        combine_s[...] = (p1 / wsum) * oh1 + (p2 / wsum) * oh2

        cnt_blk = jnp.sum(oh1 + oh2, axis=0, keepdims=True)
        psum_blk = jnp.sum(probs, axis=0, keepdims=True)
        lse = m + jnp.log(se)
        lse2_blk = jnp.sum(lse * lse, axis=0, keepdims=True)
        first = tb == 0
        cnt_s[...] = jnp.where(first, 0.0, cnt_s[...]) + cnt_blk
        psum_s[...] = jnp.where(first, 0.0, psum_s[...]) + psum_blk
        lse2_s[...] = jnp.where(first, 0.0, lse2_s[...]) + lse2_blk

    xb = x_ref[...].astype(jnp.bfloat16)
    g = jnp.dot(xb, w1_ref[0], preferred_element_type=jnp.float32)
    u = jnp.dot(xb, w3_ref[0], preferred_element_type=jnp.float32)
    hid = (g * jax.nn.sigmoid(g) * u).astype(jnp.bfloat16)
    y = jnp.dot(hid, w2_ref[0], preferred_element_type=jnp.float32)
    lane = jax.lax.broadcasted_iota(jnp.int32, (TBS, E), 1)
    c = jnp.sum(combine_s[...] * (lane == e).astype(jnp.float32),
                axis=-1, keepdims=True)
    contrib = c * y
    out_ref[...] = jnp.where(e == 0, contrib, out_ref[...] + contrib)

    @pl.when(jnp.logical_and(tb == TB - 1, e == E - 1))
    def _losses():
        counts = cnt_s[...]
        mean_probs = psum_s[...] / T
        bal_ref[...] = E * jnp.sum(counts / (T * K) * mean_probs,
                                   keepdims=True).reshape(1, 1)
        z_ref[...] = lse2_s[...] / T


@jax.jit
def kernel(x, W_gate, w1, w3, w2):
    xf = x.reshape(T, H)
    w1b = w1.astype(jnp.bfloat16)
    w3b = w3.astype(jnp.bfloat16)
    w2b = w2.astype(jnp.bfloat16)

    out, bal, z = pl.pallas_call(
        _moe_dense_kernel,
        grid=(TB, E),
        in_specs=[
            pl.BlockSpec((TBS, H), lambda tb, e: (tb, 0)),
            pl.BlockSpec((E, H), lambda tb, e: (0, 0)),
            pl.BlockSpec((1, H, F), lambda tb, e: (e, 0, 0)),
            pl.BlockSpec((1, H, F), lambda tb, e: (e, 0, 0)),
            pl.BlockSpec((1, F, H), lambda tb, e: (e, 0, 0)),
        ],
        out_specs=[
            pl.BlockSpec((TBS, H), lambda tb, e: (tb, 0)),
            pl.BlockSpec((1, 1), lambda tb, e: (0, 0)),
            pl.BlockSpec((1, 1), lambda tb, e: (0, 0)),
        ],
        out_shape=[
            jax.ShapeDtypeStruct((T, H), jnp.float32),
            jax.ShapeDtypeStruct((1, 1), jnp.float32),
            jax.ShapeDtypeStruct((1, 1), jnp.float32),
        ],
        scratch_shapes=[
            pltpu.VMEM((TBS, E), jnp.float32),
            pltpu.VMEM((1, E), jnp.float32),
            pltpu.VMEM((1, E), jnp.float32),
            pltpu.VMEM((1, 1), jnp.float32),
        ],
    )(xf, W_gate, w1b, w3b, w2b)

    return out.reshape(B, S, H), bal[0, 0], z[0, 0]
